# Initial kernel scaffold; baseline (speedup 1.0000x reference)
import jax
import jax.numpy as jnp
from jax.experimental import pallas as pl


def kernel(nodes, mask_indices, mask_values, unique_nodes_list, feature_table, W, b):
    raise NotImplementedError("write your pallas kernel here")


# speedup vs baseline: 2.2226x; 2.2226x over previous
import functools

import jax
import jax.numpy as jnp
from jax import lax
from jax.experimental import pallas as pl
from jax.experimental.pallas import tpu as pltpu
from jax.experimental.pallas import tpu_sc as plsc

NC = 2
NS = 16
LANES = 16

EB = 128
KB = 8


def _sc_body(nb, nodes_h, unique_h, rows_h, cols_h, vals_h, ftab_h, out_h,
             rows_v, cols_v, vals_v, effb, bufA, bufB, idxs, acc, sem):
    c = lax.axis_index("c")
    s = lax.axis_index("s")
    rpt = acc.shape[0] // NS
    nck = rpt // EB

    base = s * nb
    kb = rows_v.shape[0]

    zero16 = jnp.zeros((LANES,), jnp.float32)

    def zrow(i, carry):
        for j in range(8):
            bufA[i, pl.ds(j * LANES, LANES)] = zero16
        return carry

    lax.fori_loop(0, EB, zrow, 0)
    for k in range(nck):
        pltpu.sync_copy(bufA, acc.at[pl.ds(rpt * s + EB * k, EB)])

    for k in range(nck):
        pltpu.sync_copy(nodes_h.at[s, k], idxs)
        for j in range(8):
            sl = pl.ds(j * LANES, LANES)
            idxs[0, sl] = idxs[0, sl] * 2 + c
        pltpu.async_copy(ftab_h.at[idxs.at[0]], bufB, sem).wait()
        pltpu.sync_copy(bufB, out_h.at[2 + c, pl.ds(rpt * s + EB * k, EB)])

    plsc.subcore_barrier()

    def superbatch(t, carry):
        pltpu.sync_copy(rows_h.at[pl.ds(base + t * kb, kb)], rows_v)
        pltpu.sync_copy(cols_h.at[pl.ds(base + t * kb, kb)], cols_v)
        pltpu.sync_copy(vals_h.at[pl.ds((base + t * kb) * EB, kb * EB)], vals_v)

        def batch(b, cc2):
            pltpu.async_copy(unique_h.at[cols_v.at[b]], effb, sem).wait()
            for j in range(8):
                sl = pl.ds(j * LANES, LANES)
                effb[sl] = effb[sl] * 2 + c
            pltpu.async_copy(ftab_h.at[effb], bufA, sem).wait()

            def egrp(g, cc):
                wv = vals_v[pl.ds(b * EB + g * LANES, LANES)]
                for u in range(LANES):
                    w = wv[u]
                    i = g * LANES + u
                    for j in range(8):
                        sl = pl.ds(j * LANES, LANES)
                        bufA[i, sl] = bufA[i, sl] * w
                return cc

            lax.fori_loop(0, EB // LANES, egrp, 0)
            pltpu.sync_copy(bufA, acc.at[rows_v.at[b]], add=True)
            return cc2

        lax.fori_loop(0, kb, batch, 0)
        return carry

    lax.fori_loop(0, nb // kb, superbatch, 0)

    plsc.subcore_barrier()
    for k in range(nck):
        pltpu.sync_copy(acc.at[pl.ds(rpt * s + EB * k, EB)], bufB)
        pltpu.sync_copy(bufB, out_h.at[c, pl.ds(rpt * s + EB * k, EB)])


def _mm_body(x_ref, w_ref, b_ref, o_ref):
    acc = jnp.dot(x_ref[0], w_ref[0], preferred_element_type=jnp.float32)
    for k in range(1, 4):
        acc += jnp.dot(x_ref[k], w_ref[k], preferred_element_type=jnp.float32)
    acc += b_ref[...]
    o_ref[...] = acc * jax.nn.sigmoid(acc)


def kernel(nodes, mask_indices, mask_values, unique_nodes_list, feature_table, W, b):
    B = nodes.shape[0]
    NNZ = mask_values.shape[0]
    NTAB, D = feature_table.shape
    EMB = W.shape[1]
    DH = D // 2

    B_pad = ((B + NS * EB - 1) // (NS * EB)) * (NS * EB)
    NNZ_pad = ((NNZ + NS * EB * 8 - 1) // (NS * EB * 8)) * (NS * EB * 8)
    nb = NNZ_pad // NS // EB
    rpt = B_pad // NS
    nck = rpt // EB

    rows = mask_indices[0]
    cols = mask_indices[1]
    zi_e = jnp.zeros((NNZ_pad - NNZ,), jnp.int32)
    rows_p = jnp.concatenate([rows, zi_e]).reshape(NNZ_pad // EB, EB)
    cols_p = jnp.concatenate([cols, zi_e]).reshape(NNZ_pad // EB, EB)
    vals_p = jnp.concatenate(
        [mask_values, jnp.zeros((NNZ_pad - NNZ,), jnp.float32)]
    )
    nodes_p = jnp.concatenate(
        [nodes, jnp.zeros((B_pad - B,), jnp.int32)]
    ).reshape(NS, nck, 1, EB)
    ftab2 = feature_table.reshape(NTAB * 2, DH)

    mesh = plsc.VectorSubcoreMesh(
        core_axis_name="c", subcore_axis_name="s", num_cores=NC, num_subcores=NS
    )
    sc_call = pl.kernel(
        functools.partial(_sc_body, nb),
        out_type=jax.ShapeDtypeStruct((4, B_pad, DH), jnp.float32),
        mesh=mesh,
        scratch_types=[
            pltpu.VMEM((KB, EB), jnp.int32),
            pltpu.VMEM((KB, EB), jnp.int32),
            pltpu.VMEM((KB * EB,), jnp.float32),
            pltpu.VMEM((EB,), jnp.int32),
            pltpu.VMEM((EB, DH), jnp.float32),
            pltpu.VMEM((EB, DH), jnp.float32),
            pltpu.VMEM((1, EB), jnp.int32),
            pltpu.VMEM_SHARED((B_pad, DH), jnp.float32),
            pltpu.SemaphoreType.DMA,
        ],
    )
    planes = sc_call(nodes_p, unique_nodes_list, rows_p, cols_p, vals_p, ftab2)

    W4 = W.reshape(4, DH, EMB)
    b2 = b.reshape(1, EMB)
    RT = 512
    mm = pl.pallas_call(
        _mm_body,
        grid=(B_pad // RT,),
        in_specs=[
            pl.BlockSpec((4, RT, DH), lambda i: (0, i, 0)),
            pl.BlockSpec((4, DH, EMB), lambda i: (0, 0, 0)),
            pl.BlockSpec((1, EMB), lambda i: (0, 0)),
        ],
        out_specs=pl.BlockSpec((RT, EMB), lambda i: (i, 0)),
        out_shape=jax.ShapeDtypeStruct((B_pad, EMB), jnp.float32),
    )
    out = mm(planes, W4, b2)
    return out[:B]


# speedup vs baseline: 2.7728x; 1.2475x over previous
import functools

import jax
import jax.numpy as jnp
from jax import lax
from jax.experimental import pallas as pl
from jax.experimental.pallas import tpu as pltpu
from jax.experimental.pallas import tpu_sc as plsc

NC = 2
NS = 16
LANES = 16

EB = 128
KB = 8


def _sc_body(nb, nodes_h, unique_h, rows_h, cols_h, vals_h, ftab_h, out_h,
             rows_st, cols_st, vals_st, eff2, bufA, bufB, idxs, acc,
             semA, semB, semG):
    c = lax.axis_index("c")
    s = lax.axis_index("s")
    rpt = acc.shape[0] // NS
    nck = rpt // EB

    base = s * nb
    kb = rows_st.shape[0]

    zero16 = jnp.zeros((LANES,), jnp.float32)

    def zrow(i, carry):
        for j in range(8):
            bufA[i, pl.ds(j * LANES, LANES)] = zero16
        return carry

    lax.fori_loop(0, EB, zrow, 0)
    for k in range(nck):
        pltpu.sync_copy(bufA, acc.at[pl.ds(rpt * s + EB * k, EB)])

    for k in range(nck):
        pltpu.sync_copy(nodes_h.at[s, k], idxs)
        for j in range(8):
            sl = pl.ds(j * LANES, LANES)
            idxs[0, sl] = idxs[0, sl] * 2 + c
        pltpu.async_copy(ftab_h.at[idxs.at[0]], bufB, semG).wait()
        pltpu.sync_copy(bufB, out_h.at[2 + c, pl.ds(rpt * s + EB * k, EB)])

    def effsb(t, carry):
        pltpu.sync_copy(cols_h.at[pl.ds(base + t * kb, kb)], cols_st)
        for tb in range(8):
            pltpu.async_copy(unique_h.at[cols_st.at[tb]],
                             eff2.at[t * 8 + tb], semG)
        for tb in range(8):
            pltpu.make_async_copy(unique_h.at[cols_st.at[tb]],
                                  eff2.at[t * 8 + tb], semG).wait()
        return carry

    lax.fori_loop(0, nb // 8, effsb, 0)

    def efffix(r, carry):
        for j in range(8):
            sl = pl.ds(j * LANES, LANES)
            eff2[r, sl] = eff2[r, sl] * 2 + c
        return carry

    lax.fori_loop(0, nb, efffix, 0)

    pltpu.async_copy(ftab_h.at[eff2.at[0]], bufA, semA)

    plsc.subcore_barrier()

    def piter(i, carry):
        b0 = 2 * i
        li = lax.rem(i, 4)
        lb0 = 2 * li

        @pl.when(li == 0)
        def _stage():
            t = i // 4
            pltpu.sync_copy(rows_h.at[pl.ds(base + t * kb, kb)], rows_st)
            pltpu.sync_copy(vals_h.at[pl.ds((base + t * kb) * EB, kb * EB)],
                            vals_st)

        pltpu.async_copy(ftab_h.at[eff2.at[b0 + 1]], bufB, semB)
        pltpu.make_async_copy(ftab_h.at[eff2.at[b0]], bufA, semA).wait()

        def egrpA(g, cc):
            wv = vals_st[pl.ds(lb0 * EB + g * LANES, LANES)]
            for u in range(LANES):
                w = wv[u]
                i2 = g * LANES + u
                for j in range(8):
                    sl = pl.ds(j * LANES, LANES)
                    bufA[i2, sl] = bufA[i2, sl] * w
            return cc

        lax.fori_loop(0, EB // LANES, egrpA, 0)
        pltpu.async_copy(bufA, acc.at[rows_st.at[lb0]], semA, add=True)
        pltpu.make_async_copy(ftab_h.at[eff2.at[b0]], bufA, semA).wait()

        @pl.when(i < nb // 2 - 1)
        def _next():
            pltpu.async_copy(ftab_h.at[eff2.at[b0 + 2]], bufA, semA)

        pltpu.make_async_copy(ftab_h.at[eff2.at[b0 + 1]], bufB, semB).wait()

        def egrpB(g, cc):
            wv = vals_st[pl.ds((lb0 + 1) * EB + g * LANES, LANES)]
            for u in range(LANES):
                w = wv[u]
                i2 = g * LANES + u
                for j in range(8):
                    sl = pl.ds(j * LANES, LANES)
                    bufB[i2, sl] = bufB[i2, sl] * w
            return cc

        lax.fori_loop(0, EB // LANES, egrpB, 0)
        pltpu.async_copy(bufB, acc.at[rows_st.at[lb0 + 1]], semB, add=True)
        pltpu.make_async_copy(ftab_h.at[eff2.at[b0 + 1]], bufB, semB).wait()
        return carry

    lax.fori_loop(0, nb // 2, piter, 0)

    plsc.subcore_barrier()
    for k in range(nck):
        pltpu.sync_copy(acc.at[pl.ds(rpt * s + EB * k, EB)], bufB)
        pltpu.sync_copy(bufB, out_h.at[c, pl.ds(rpt * s + EB * k, EB)])


def _mm_body(x_ref, w_ref, b_ref, o_ref):
    acc = jnp.dot(x_ref[0], w_ref[0], preferred_element_type=jnp.float32)
    for k in range(1, 4):
        acc += jnp.dot(x_ref[k], w_ref[k], preferred_element_type=jnp.float32)
    acc += b_ref[...]
    o_ref[...] = acc * jax.nn.sigmoid(acc)


def kernel(nodes, mask_indices, mask_values, unique_nodes_list, feature_table, W, b):
    B = nodes.shape[0]
    NNZ = mask_values.shape[0]
    NTAB, D = feature_table.shape
    EMB = W.shape[1]
    DH = D // 2

    B_pad = ((B + NS * EB - 1) // (NS * EB)) * (NS * EB)
    NNZ_pad = ((NNZ + NS * EB * 8 - 1) // (NS * EB * 8)) * (NS * EB * 8)
    nb = NNZ_pad // NS // EB
    rpt = B_pad // NS
    nck = rpt // EB

    rows = mask_indices[0]
    cols = mask_indices[1]
    zi_e = jnp.zeros((NNZ_pad - NNZ,), jnp.int32)
    rows_p = jnp.concatenate([rows, zi_e]).reshape(NNZ_pad // EB, EB)
    cols_p = jnp.concatenate([cols, zi_e]).reshape(NNZ_pad // EB, EB)
    vals_p = jnp.concatenate(
        [mask_values, jnp.zeros((NNZ_pad - NNZ,), jnp.float32)]
    )
    nodes_p = jnp.concatenate(
        [nodes, jnp.zeros((B_pad - B,), jnp.int32)]
    ).reshape(NS, nck, 1, EB)
    ftab2 = feature_table.reshape(NTAB * 2, DH)

    mesh = plsc.VectorSubcoreMesh(
        core_axis_name="c", subcore_axis_name="s", num_cores=NC, num_subcores=NS
    )
    sc_call = pl.kernel(
        functools.partial(_sc_body, nb),
        out_type=jax.ShapeDtypeStruct((4, B_pad, DH), jnp.float32),
        mesh=mesh,
        scratch_types=[
            pltpu.VMEM((KB, EB), jnp.int32),
            pltpu.VMEM((KB, EB), jnp.int32),
            pltpu.VMEM((KB * EB,), jnp.float32),
            pltpu.VMEM((nb, EB), jnp.int32),
            pltpu.VMEM((EB, DH), jnp.float32),
            pltpu.VMEM((EB, DH), jnp.float32),
            pltpu.VMEM((1, EB), jnp.int32),
            pltpu.VMEM_SHARED((B_pad, DH), jnp.float32),
            pltpu.SemaphoreType.DMA,
            pltpu.SemaphoreType.DMA,
            pltpu.SemaphoreType.DMA,
        ],
    )
    planes = sc_call(nodes_p, unique_nodes_list, rows_p, cols_p, vals_p, ftab2)

    W4 = W.reshape(4, DH, EMB)
    b2 = b.reshape(1, EMB)
    RT = 512
    mm = pl.pallas_call(
        _mm_body,
        grid=(B_pad // RT,),
        in_specs=[
            pl.BlockSpec((4, RT, DH), lambda i: (0, i, 0)),
            pl.BlockSpec((4, DH, EMB), lambda i: (0, 0, 0)),
            pl.BlockSpec((1, EMB), lambda i: (0, 0)),
        ],
        out_specs=pl.BlockSpec((RT, EMB), lambda i: (i, 0)),
        out_shape=jax.ShapeDtypeStruct((B_pad, EMB), jnp.float32),
    )
    out = mm(planes, W4, b2)
    return out[:B]


# speedup vs baseline: 3.0099x; 1.0855x over previous
import functools

import jax
import jax.numpy as jnp
from jax import lax
from jax.experimental import pallas as pl
from jax.experimental.pallas import tpu as pltpu
from jax.experimental.pallas import tpu_sc as plsc

NC = 2
NS = 16
LANES = 16

EB = 128
KB = 8


def _sc_body(nb, nodes_h, uniq_h, rows_h, cols_h, vals_h, ftab_h, out_h,
             rows_st, colsA_st, colsB_st, vals_st, uniq_v,
             effbA, effbB, bufA, bufB, idxs, acc, semA, semB, semG):
    c = lax.axis_index("c")
    s = lax.axis_index("s")
    rpt = acc.shape[0] // NS
    nck = rpt // EB

    base = s * nb
    kb = rows_st.shape[0]
    nsb = nb // kb

    pltpu.sync_copy(uniq_h, uniq_v)

    def compute_eff(cols_ref, lb, dst):
        for g in range(EB // LANES):
            sl = pl.ds(g * LANES, LANES)
            cv = cols_ref[lb, sl]
            pair = plsc.load_gather(uniq_v, [lax.shift_right_logical(cv, 1)])
            sh = (cv & 1) << 4
            u = lax.shift_right_logical(pair, sh) & 0xFFFF
            dst[sl] = u * 2 + c

    zero16 = jnp.zeros((LANES,), jnp.float32)

    def zrow(i, carry):
        for j in range(8):
            bufA[i, pl.ds(j * LANES, LANES)] = zero16
        return carry

    lax.fori_loop(0, EB, zrow, 0)
    for k in range(nck):
        pltpu.sync_copy(bufA, acc.at[pl.ds(rpt * s + EB * k, EB)])

    for k in range(nck):
        pltpu.sync_copy(nodes_h.at[s, k], idxs)
        for j in range(8):
            sl = pl.ds(j * LANES, LANES)
            idxs[0, sl] = idxs[0, sl] * 2 + c
        pltpu.async_copy(ftab_h.at[idxs.at[0]], bufB, semG).wait()
        pltpu.sync_copy(bufB, out_h.at[2 + c, pl.ds(rpt * s + EB * k, EB)])

    pltpu.sync_copy(cols_h.at[pl.ds(base, kb)], colsA_st)
    compute_eff(colsA_st, 0, effbA)
    pltpu.async_copy(ftab_h.at[effbA], bufA, semA)

    plsc.subcore_barrier()

    def piter(i, carry):
        li = lax.rem(i, 4)
        t = i // 4
        p = lax.rem(t, 2)
        lb0 = 2 * li

        @pl.when(li == 0)
        def _stage():
            pltpu.sync_copy(rows_h.at[pl.ds(base + t * kb, kb)], rows_st)
            pltpu.sync_copy(vals_h.at[pl.ds((base + t * kb) * EB, kb * EB)],
                            vals_st)

        @pl.when(p == 0)
        def _eb0():
            compute_eff(colsA_st, lb0 + 1, effbB)

        @pl.when(p == 1)
        def _eb1():
            compute_eff(colsB_st, lb0 + 1, effbB)

        pltpu.async_copy(ftab_h.at[effbB], bufB, semB)

        @pl.when((li == 2) & (t + 1 < nsb))
        def _pf():
            @pl.when(p == 0)
            def _pf0():
                pltpu.sync_copy(cols_h.at[pl.ds(base + (t + 1) * kb, kb)],
                                colsB_st)

            @pl.when(p == 1)
            def _pf1():
                pltpu.sync_copy(cols_h.at[pl.ds(base + (t + 1) * kb, kb)],
                                colsA_st)

        pltpu.make_async_copy(ftab_h.at[effbA], bufA, semA).wait()

        def egrpA(g, cc):
            wv = vals_st[pl.ds(lb0 * EB + g * LANES, LANES)]
            for u in range(LANES):
                w = wv[u]
                i2 = g * LANES + u
                for j in range(8):
                    sl = pl.ds(j * LANES, LANES)
                    bufA[i2, sl] = bufA[i2, sl] * w
            return cc

        lax.fori_loop(0, EB // LANES, egrpA, 0)
        pltpu.async_copy(bufA, acc.at[rows_st.at[lb0]], semA, add=True)
        pltpu.make_async_copy(ftab_h.at[effbA], bufA, semA).wait()

        @pl.when(i < nb // 2 - 1)
        def _next():
            @pl.when(li < 3)
            def _n_same():
                @pl.when(p == 0)
                def _ns0():
                    compute_eff(colsA_st, lb0 + 2, effbA)

                @pl.when(p == 1)
                def _ns1():
                    compute_eff(colsB_st, lb0 + 2, effbA)

            @pl.when(li == 3)
            def _n_cross():
                @pl.when(p == 0)
                def _nc0():
                    compute_eff(colsB_st, 0, effbA)

                @pl.when(p == 1)
                def _nc1():
                    compute_eff(colsA_st, 0, effbA)

            pltpu.async_copy(ftab_h.at[effbA], bufA, semA)

        pltpu.make_async_copy(ftab_h.at[effbB], bufB, semB).wait()

        def egrpB(g, cc):
            wv = vals_st[pl.ds((lb0 + 1) * EB + g * LANES, LANES)]
            for u in range(LANES):
                w = wv[u]
                i2 = g * LANES + u
                for j in range(8):
                    sl = pl.ds(j * LANES, LANES)
                    bufB[i2, sl] = bufB[i2, sl] * w
            return cc

        lax.fori_loop(0, EB // LANES, egrpB, 0)
        pltpu.async_copy(bufB, acc.at[rows_st.at[lb0 + 1]], semB, add=True)
        pltpu.make_async_copy(ftab_h.at[effbB], bufB, semB).wait()
        return carry

    lax.fori_loop(0, nb // 2, piter, 0)

    plsc.subcore_barrier()
    for k in range(nck):
        pltpu.sync_copy(acc.at[pl.ds(rpt * s + EB * k, EB)], bufB)
        pltpu.sync_copy(bufB, out_h.at[c, pl.ds(rpt * s + EB * k, EB)])


def _mm_body(x_ref, w_ref, b_ref, o_ref):
    acc = jnp.dot(x_ref[0], w_ref[0], preferred_element_type=jnp.float32)
    for k in range(1, 4):
        acc += jnp.dot(x_ref[k], w_ref[k], preferred_element_type=jnp.float32)
    acc += b_ref[...]
    o_ref[...] = acc * jax.nn.sigmoid(acc)


def kernel(nodes, mask_indices, mask_values, unique_nodes_list, feature_table, W, b):
    B = nodes.shape[0]
    NNZ = mask_values.shape[0]
    NTAB, D = feature_table.shape
    EMB = W.shape[1]
    DH = D // 2

    B_pad = ((B + NS * EB - 1) // (NS * EB)) * (NS * EB)
    NNZ_pad = ((NNZ + NS * EB * 8 - 1) // (NS * EB * 8)) * (NS * EB * 8)
    nb = NNZ_pad // NS // EB
    rpt = B_pad // NS
    nck = rpt // EB

    rows = mask_indices[0]
    cols = mask_indices[1]
    zi_e = jnp.zeros((NNZ_pad - NNZ,), jnp.int32)
    rows_p = jnp.concatenate([rows, zi_e]).reshape(NNZ_pad // EB, EB)
    cols_p = jnp.concatenate([cols, zi_e]).reshape(NNZ_pad // EB, EB)
    vals_p = jnp.concatenate(
        [mask_values, jnp.zeros((NNZ_pad - NNZ,), jnp.float32)]
    )
    nodes_p = jnp.concatenate(
        [nodes, jnp.zeros((B_pad - B,), jnp.int32)]
    ).reshape(NS, nck, 1, EB)
    ftab2 = feature_table.reshape(NTAB * 2, DH)

    U = unique_nodes_list.shape[0]
    U_pad = ((U + 2 * LANES - 1) // (2 * LANES)) * (2 * LANES)
    up = jnp.concatenate(
        [unique_nodes_list, jnp.zeros((U_pad - U,), jnp.int32)]
    ).astype(jnp.uint32)
    uniq16 = lax.bitcast_convert_type(up[0::2] | (up[1::2] << 16), jnp.int32)

    mesh = plsc.VectorSubcoreMesh(
        core_axis_name="c", subcore_axis_name="s", num_cores=NC, num_subcores=NS
    )
    sc_call = pl.kernel(
        functools.partial(_sc_body, nb),
        out_type=jax.ShapeDtypeStruct((4, B_pad, DH), jnp.float32),
        mesh=mesh,
        scratch_types=[
            pltpu.VMEM((KB, EB), jnp.int32),
            pltpu.VMEM((KB, EB), jnp.int32),
            pltpu.VMEM((KB, EB), jnp.int32),
            pltpu.VMEM((KB * EB,), jnp.float32),
            pltpu.VMEM((U_pad // 2,), jnp.int32),
            pltpu.VMEM((EB,), jnp.int32),
            pltpu.VMEM((EB,), jnp.int32),
            pltpu.VMEM((EB, DH), jnp.float32),
            pltpu.VMEM((EB, DH), jnp.float32),
            pltpu.VMEM((1, EB), jnp.int32),
            pltpu.VMEM_SHARED((B_pad, DH), jnp.float32),
            pltpu.SemaphoreType.DMA,
            pltpu.SemaphoreType.DMA,
            pltpu.SemaphoreType.DMA,
        ],
        compiler_params=pltpu.CompilerParams(needs_layout_passes=False),
    )
    planes = sc_call(nodes_p, uniq16, rows_p, cols_p, vals_p, ftab2)

    W4 = W.reshape(4, DH, EMB)
    b2 = b.reshape(1, EMB)
    RT = 512
    mm = pl.pallas_call(
        _mm_body,
        grid=(B_pad // RT,),
        in_specs=[
            pl.BlockSpec((4, RT, DH), lambda i: (0, i, 0)),
            pl.BlockSpec((4, DH, EMB), lambda i: (0, 0, 0)),
            pl.BlockSpec((1, EMB), lambda i: (0, 0)),
        ],
        out_specs=pl.BlockSpec((RT, EMB), lambda i: (i, 0)),
        out_shape=jax.ShapeDtypeStruct((B_pad, EMB), jnp.float32),
    )
    out = mm(planes, W4, b2)
    return out[:B]


# speedup vs baseline: 3.1173x; 1.0357x over previous
import functools

import jax
import jax.numpy as jnp
from jax import lax
from jax.experimental import pallas as pl
from jax.experimental.pallas import tpu as pltpu
from jax.experimental.pallas import tpu_sc as plsc

NC = 2
NS = 16
LANES = 16

EB = 128
KB = 8


def _sc_body(nb, nodes_h, uniq_h, rows_h, cols_h, vals_h, ftab_h, out_h,
             rows_st, colsA_st, colsB_st, vals_st, uniq_v,
             effbA, effbB, bufA, bufB, idxs, acc, semA, semB, semG):
    c = lax.axis_index("c")
    s = lax.axis_index("s")
    rpt = acc.shape[0] // NS
    nck = rpt // EB

    base = s * nb
    kb = rows_st.shape[0]
    nsb = nb // kb

    pltpu.sync_copy(uniq_h, uniq_v)

    def compute_eff(cols_ref, lb, dst):
        for g in range(EB // LANES):
            sl = pl.ds(g * LANES, LANES)
            cv = cols_ref[lb, sl]
            pair = plsc.load_gather(uniq_v, [lax.shift_right_logical(cv, 1)])
            sh = (cv & 1) << 4
            u = lax.shift_right_logical(pair, sh) & 0xFFFF
            dst[sl] = u * 2 + c

    zero16 = jnp.zeros((LANES,), jnp.float32)

    def zrow(i, carry):
        for j in range(8):
            bufA[i, pl.ds(j * LANES, LANES)] = zero16
        return carry

    lax.fori_loop(0, EB, zrow, 0)
    for k in range(nck):
        pltpu.sync_copy(bufA, acc.at[pl.ds(rpt * s + EB * k, EB)])

    for k in range(nck):
        pltpu.sync_copy(nodes_h.at[s, k], idxs)
        for j in range(8):
            sl = pl.ds(j * LANES, LANES)
            idxs[0, sl] = idxs[0, sl] * 2 + c
        pltpu.async_copy(ftab_h.at[idxs.at[0]], bufB, semG).wait()
        pltpu.sync_copy(bufB, out_h.at[2 + c, pl.ds(rpt * s + EB * k, EB)])

    pltpu.sync_copy(cols_h.at[pl.ds(base, kb)], colsA_st)
    compute_eff(colsA_st, 0, effbA)
    pltpu.async_copy(ftab_h.at[effbA], bufA, semA)

    plsc.subcore_barrier()

    def piter(i, carry):
        li = lax.rem(i, 4)
        t = i // 4
        p = lax.rem(t, 2)
        lb0 = 2 * li

        @pl.when(li == 0)
        def _stage():
            pltpu.sync_copy(rows_h.at[pl.ds(base + t * kb, kb)], rows_st)
            pltpu.sync_copy(vals_h.at[pl.ds((base + t * kb) * EB, kb * EB)],
                            vals_st)

        @pl.when(p == 0)
        def _eb0():
            compute_eff(colsA_st, lb0 + 1, effbB)

        @pl.when(p == 1)
        def _eb1():
            compute_eff(colsB_st, lb0 + 1, effbB)

        pltpu.async_copy(ftab_h.at[effbB], bufB, semB)

        @pl.when((li == 2) & (t + 1 < nsb))
        def _pf():
            @pl.when(p == 0)
            def _pf0():
                pltpu.sync_copy(cols_h.at[pl.ds(base + (t + 1) * kb, kb)],
                                colsB_st)

            @pl.when(p == 1)
            def _pf1():
                pltpu.sync_copy(cols_h.at[pl.ds(base + (t + 1) * kb, kb)],
                                colsA_st)

        pltpu.make_async_copy(ftab_h.at[effbA], bufA, semA).wait()

        def egrpA(g, cc):
            wv = vals_st[pl.ds(lb0 * EB + g * LANES, LANES)]
            for u in range(LANES):
                w = wv[u]
                i2 = g * LANES + u
                for j in range(8):
                    sl = pl.ds(j * LANES, LANES)
                    bufA[i2, sl] = bufA[i2, sl] * w
            return cc

        lax.fori_loop(0, EB // LANES, egrpA, 0)
        pltpu.async_copy(bufA, acc.at[rows_st.at[lb0]], semA, add=True)
        pltpu.make_async_copy(ftab_h.at[effbA], bufA, semA).wait()

        @pl.when(i < nb // 2 - 1)
        def _next():
            @pl.when(li < 3)
            def _n_same():
                @pl.when(p == 0)
                def _ns0():
                    compute_eff(colsA_st, lb0 + 2, effbA)

                @pl.when(p == 1)
                def _ns1():
                    compute_eff(colsB_st, lb0 + 2, effbA)

            @pl.when(li == 3)
            def _n_cross():
                @pl.when(p == 0)
                def _nc0():
                    compute_eff(colsB_st, 0, effbA)

                @pl.when(p == 1)
                def _nc1():
                    compute_eff(colsA_st, 0, effbA)

            pltpu.async_copy(ftab_h.at[effbA], bufA, semA)

        pltpu.make_async_copy(ftab_h.at[effbB], bufB, semB).wait()

        def egrpB(g, cc):
            wv = vals_st[pl.ds((lb0 + 1) * EB + g * LANES, LANES)]
            for u in range(LANES):
                w = wv[u]
                i2 = g * LANES + u
                for j in range(8):
                    sl = pl.ds(j * LANES, LANES)
                    bufB[i2, sl] = bufB[i2, sl] * w
            return cc

        lax.fori_loop(0, EB // LANES, egrpB, 0)
        pltpu.async_copy(bufB, acc.at[rows_st.at[lb0 + 1]], semB, add=True)
        pltpu.make_async_copy(ftab_h.at[effbB], bufB, semB).wait()
        return carry

    lax.fori_loop(0, nb // 2, piter, 0)

    plsc.subcore_barrier()
    for k in range(nck):
        pltpu.sync_copy(acc.at[pl.ds(rpt * s + EB * k, EB)], bufB)
        pltpu.sync_copy(bufB, out_h.at[c, pl.ds(rpt * s + EB * k, EB)])


def _prep_body(mi_ref, mv_ref, r_ref, c_ref, v_ref):
    n = mi_ref.shape[1]
    pad = r_ref.shape[0] - n
    r_ref[pl.ds(0, n)] = mi_ref[0, :]
    c_ref[pl.ds(0, n)] = mi_ref[1, :]
    v_ref[pl.ds(0, n)] = mv_ref[...]
    r_ref[pl.ds(n, pad)] = jnp.zeros((pad,), jnp.int32)
    c_ref[pl.ds(n, pad)] = jnp.zeros((pad,), jnp.int32)
    v_ref[pl.ds(n, pad)] = jnp.zeros((pad,), jnp.float32)


def _mm_body(x_ref, w_ref, b_ref, o_ref):
    acc = jnp.dot(x_ref[0], w_ref[0], preferred_element_type=jnp.float32)
    for k in range(1, 4):
        acc += jnp.dot(x_ref[k], w_ref[k], preferred_element_type=jnp.float32)
    acc += b_ref[...]
    o_ref[...] = acc * jax.nn.sigmoid(acc)


def kernel(nodes, mask_indices, mask_values, unique_nodes_list, feature_table, W, b):
    B = nodes.shape[0]
    NNZ = mask_values.shape[0]
    NTAB, D = feature_table.shape
    EMB = W.shape[1]
    DH = D // 2

    B_pad = ((B + NS * EB - 1) // (NS * EB)) * (NS * EB)
    NNZ_pad = ((NNZ + NS * EB * 8 - 1) // (NS * EB * 8)) * (NS * EB * 8)
    nb = NNZ_pad // NS // EB
    rpt = B_pad // NS
    nck = rpt // EB

    prep = pl.pallas_call(
        _prep_body,
        out_shape=(
            jax.ShapeDtypeStruct((NNZ_pad,), jnp.int32),
            jax.ShapeDtypeStruct((NNZ_pad,), jnp.int32),
            jax.ShapeDtypeStruct((NNZ_pad,), jnp.float32),
        ),
    )
    rows_f, cols_f, vals_p = prep(mask_indices, mask_values)
    rows_p = rows_f.reshape(NNZ_pad // EB, EB)
    cols_p = cols_f.reshape(NNZ_pad // EB, EB)
    nodes_p = jnp.concatenate(
        [nodes, jnp.zeros((B_pad - B,), jnp.int32)]
    ).reshape(NS, nck, 1, EB)
    ftab2 = feature_table.reshape(NTAB * 2, DH)

    U = unique_nodes_list.shape[0]
    U_pad = ((U + 2 * LANES - 1) // (2 * LANES)) * (2 * LANES)
    up = jnp.concatenate(
        [unique_nodes_list, jnp.zeros((U_pad - U,), jnp.int32)]
    ).astype(jnp.uint16)
    uniq16 = lax.bitcast_convert_type(up.reshape(U_pad // 2, 2), jnp.int32)

    mesh = plsc.VectorSubcoreMesh(
        core_axis_name="c", subcore_axis_name="s", num_cores=NC, num_subcores=NS
    )
    sc_call = pl.kernel(
        functools.partial(_sc_body, nb),
        out_type=jax.ShapeDtypeStruct((4, B_pad, DH), jnp.float32),
        mesh=mesh,
        scratch_types=[
            pltpu.VMEM((KB, EB), jnp.int32),
            pltpu.VMEM((KB, EB), jnp.int32),
            pltpu.VMEM((KB, EB), jnp.int32),
            pltpu.VMEM((KB * EB,), jnp.float32),
            pltpu.VMEM((U_pad // 2,), jnp.int32),
            pltpu.VMEM((EB,), jnp.int32),
            pltpu.VMEM((EB,), jnp.int32),
            pltpu.VMEM((EB, DH), jnp.float32),
            pltpu.VMEM((EB, DH), jnp.float32),
            pltpu.VMEM((1, EB), jnp.int32),
            pltpu.VMEM_SHARED((B_pad, DH), jnp.float32),
            pltpu.SemaphoreType.DMA,
            pltpu.SemaphoreType.DMA,
            pltpu.SemaphoreType.DMA,
        ],
        compiler_params=pltpu.CompilerParams(needs_layout_passes=False),
    )
    planes = sc_call(nodes_p, uniq16, rows_p, cols_p, vals_p, ftab2)

    W4 = W.reshape(4, DH, EMB)
    b2 = b.reshape(1, EMB)
    RT = 512
    mm = pl.pallas_call(
        _mm_body,
        grid=(B_pad // RT,),
        in_specs=[
            pl.BlockSpec((4, RT, DH), lambda i: (0, i, 0)),
            pl.BlockSpec((4, DH, EMB), lambda i: (0, 0, 0)),
            pl.BlockSpec((1, EMB), lambda i: (0, 0)),
        ],
        out_specs=pl.BlockSpec((RT, EMB), lambda i: (i, 0)),
        out_shape=jax.ShapeDtypeStruct((B_pad, EMB), jnp.float32),
    )
    out = mm(planes, W4, b2)
    return out[:B]


# speedup vs baseline: 3.1786x; 1.0197x over previous
import functools

import jax
import jax.numpy as jnp
from jax import lax
from jax.experimental import pallas as pl
from jax.experimental.pallas import tpu as pltpu
from jax.experimental.pallas import tpu_sc as plsc

NC = 2
NS = 16
LANES = 16

EB = 128
KB = 8


def _sc_body(nb, nodes_h, uniq_h, rows_h, cols_h, vals_h, ftab_h, out_h,
             rows_st, colsA_st, colsB_st, vals_st, uniq_v,
             effbA, effbB, bufA, bufB, idxs, acc, semA, semB, semG):
    c = lax.axis_index("c")
    s = lax.axis_index("s")
    rpt = acc.shape[0] // NS
    nck = rpt // EB

    base = s * nb
    kb = rows_st.shape[0]
    nsb = nb // kb

    pltpu.sync_copy(uniq_h, uniq_v)

    def compute_eff(cols_ref, lb, dst):
        for g in range(EB // LANES):
            sl = pl.ds(g * LANES, LANES)
            cv = cols_ref[lb, sl]
            pair = plsc.load_gather(uniq_v, [lax.shift_right_logical(cv, 1)])
            sh = (cv & 1) << 4
            u = lax.shift_right_logical(pair, sh) & 0xFFFF
            dst[sl] = u * 2 + c

    zero16 = jnp.zeros((LANES,), jnp.float32)

    def zrow(i, carry):
        for j in range(8):
            bufA[i, pl.ds(j * LANES, LANES)] = zero16
        return carry

    lax.fori_loop(0, EB, zrow, 0)
    for k in range(nck):
        pltpu.sync_copy(bufA, acc.at[pl.ds(rpt * s + EB * k, EB)])

    for k in range(nck):
        pltpu.sync_copy(nodes_h.at[s, k], idxs)
        for j in range(8):
            sl = pl.ds(j * LANES, LANES)
            idxs[0, sl] = idxs[0, sl] * 2 + c
        pltpu.async_copy(ftab_h.at[idxs.at[0]], bufB, semG).wait()
        pltpu.sync_copy(bufB, out_h.at[2 + c, pl.ds(rpt * s + EB * k, EB)])

    pltpu.sync_copy(cols_h.at[pl.ds(base, kb)], colsA_st)
    compute_eff(colsA_st, 0, effbA)
    pltpu.async_copy(ftab_h.at[effbA], bufA, semA)

    plsc.subcore_barrier()

    def piter(i, carry):
        li = lax.rem(i, 4)
        t = i // 4
        p = lax.rem(t, 2)
        lb0 = 2 * li

        @pl.when(li == 0)
        def _stage():
            pltpu.sync_copy(rows_h.at[pl.ds(base + t * kb, kb)], rows_st)
            pltpu.sync_copy(vals_h.at[pl.ds((base + t * kb) * EB, kb * EB)],
                            vals_st)

        @pl.when(p == 0)
        def _eb0():
            compute_eff(colsA_st, lb0 + 1, effbB)

        @pl.when(p == 1)
        def _eb1():
            compute_eff(colsB_st, lb0 + 1, effbB)

        pltpu.async_copy(ftab_h.at[effbB], bufB, semB)

        @pl.when((li == 2) & (t + 1 < nsb))
        def _pf():
            @pl.when(p == 0)
            def _pf0():
                pltpu.sync_copy(cols_h.at[pl.ds(base + (t + 1) * kb, kb)],
                                colsB_st)

            @pl.when(p == 1)
            def _pf1():
                pltpu.sync_copy(cols_h.at[pl.ds(base + (t + 1) * kb, kb)],
                                colsA_st)

        pltpu.make_async_copy(ftab_h.at[effbA], bufA, semA).wait()

        def egrpA(g, cc):
            wv = vals_st[pl.ds(lb0 * EB + g * LANES, LANES)]
            for u in range(LANES):
                w = wv[u]
                i2 = g * LANES + u
                for j in range(8):
                    sl = pl.ds(j * LANES, LANES)
                    bufA[i2, sl] = bufA[i2, sl] * w
            return cc

        lax.fori_loop(0, EB // LANES, egrpA, 0)
        pltpu.async_copy(bufA, acc.at[rows_st.at[lb0]], semA, add=True)
        pltpu.make_async_copy(ftab_h.at[effbA], bufA, semA).wait()

        @pl.when(i < nb // 2 - 1)
        def _next():
            @pl.when(li < 3)
            def _n_same():
                @pl.when(p == 0)
                def _ns0():
                    compute_eff(colsA_st, lb0 + 2, effbA)

                @pl.when(p == 1)
                def _ns1():
                    compute_eff(colsB_st, lb0 + 2, effbA)

            @pl.when(li == 3)
            def _n_cross():
                @pl.when(p == 0)
                def _nc0():
                    compute_eff(colsB_st, 0, effbA)

                @pl.when(p == 1)
                def _nc1():
                    compute_eff(colsA_st, 0, effbA)

            pltpu.async_copy(ftab_h.at[effbA], bufA, semA)

        pltpu.make_async_copy(ftab_h.at[effbB], bufB, semB).wait()

        def egrpB(g, cc):
            wv = vals_st[pl.ds((lb0 + 1) * EB + g * LANES, LANES)]
            for u in range(LANES):
                w = wv[u]
                i2 = g * LANES + u
                for j in range(8):
                    sl = pl.ds(j * LANES, LANES)
                    bufB[i2, sl] = bufB[i2, sl] * w
            return cc

        lax.fori_loop(0, EB // LANES, egrpB, 0)
        pltpu.async_copy(bufB, acc.at[rows_st.at[lb0 + 1]], semB, add=True)
        pltpu.make_async_copy(ftab_h.at[effbB], bufB, semB).wait()
        return carry

    lax.fori_loop(0, nb // 2, piter, 0)

    plsc.subcore_barrier()
    for k in range(nck):
        pltpu.sync_copy(acc.at[pl.ds(rpt * s + EB * k, EB)], bufB)
        pltpu.sync_copy(bufB, out_h.at[c, pl.ds(rpt * s + EB * k, EB)])


def _prep_body(mi_ref, mv_ref, r_ref, c_ref, v_ref):
    n = mi_ref.shape[1]
    pad = r_ref.shape[0] - n
    r_ref[pl.ds(0, n)] = mi_ref[0, :]
    c_ref[pl.ds(0, n)] = mi_ref[1, :]
    v_ref[pl.ds(0, n)] = mv_ref[...]
    r_ref[pl.ds(n, pad)] = jnp.zeros((pad,), jnp.int32)
    c_ref[pl.ds(n, pad)] = jnp.zeros((pad,), jnp.int32)
    v_ref[pl.ds(n, pad)] = jnp.zeros((pad,), jnp.float32)


def _mm_body(x_ref, w_ref, b_ref, o_ref):
    acc = jnp.dot(x_ref[0], w_ref[0], preferred_element_type=jnp.float32)
    for k in range(1, 4):
        acc += jnp.dot(x_ref[k], w_ref[k], preferred_element_type=jnp.float32)
    acc += b_ref[...]
    o_ref[...] = acc * jax.nn.sigmoid(acc)


def kernel(nodes, mask_indices, mask_values, unique_nodes_list, feature_table, W, b):
    B = nodes.shape[0]
    NNZ = mask_values.shape[0]
    NTAB, D = feature_table.shape
    EMB = W.shape[1]
    DH = D // 2

    B_pad = ((B + NS * EB - 1) // (NS * EB)) * (NS * EB)
    NNZ_pad = ((NNZ + NS * EB * 8 - 1) // (NS * EB * 8)) * (NS * EB * 8)
    nb = NNZ_pad // NS // EB
    rpt = B_pad // NS
    nck = rpt // EB

    prep = pl.pallas_call(
        _prep_body,
        out_shape=(
            jax.ShapeDtypeStruct((NNZ_pad,), jnp.int32),
            jax.ShapeDtypeStruct((NNZ_pad,), jnp.int32),
            jax.ShapeDtypeStruct((NNZ_pad,), jnp.float32),
        ),
    )
    rows_f, cols_f, vals_p = prep(mask_indices, mask_values)
    rows_p = rows_f.reshape(NNZ_pad // EB, EB)
    cols_p = cols_f.reshape(NNZ_pad // EB, EB)
    nodes_p = jnp.concatenate(
        [nodes, jnp.zeros((B_pad - B,), jnp.int32)]
    ).reshape(NS, nck, 1, EB)
    ftab2 = feature_table.reshape(NTAB * 2, DH)

    U = unique_nodes_list.shape[0]
    U_pad = ((U + 2 * LANES - 1) // (2 * LANES)) * (2 * LANES)
    up = jnp.concatenate(
        [unique_nodes_list, jnp.zeros((U_pad - U,), jnp.int32)]
    ).astype(jnp.uint16)
    uniq16 = lax.bitcast_convert_type(up.reshape(U_pad // 2, 2), jnp.int32)

    mesh = plsc.VectorSubcoreMesh(
        core_axis_name="c", subcore_axis_name="s", num_cores=NC, num_subcores=NS
    )
    sc_call = pl.kernel(
        functools.partial(_sc_body, nb),
        out_type=jax.ShapeDtypeStruct((4, B_pad, DH), jnp.float32),
        mesh=mesh,
        scratch_types=[
            pltpu.VMEM((KB, EB), jnp.int32),
            pltpu.VMEM((KB, EB), jnp.int32),
            pltpu.VMEM((KB, EB), jnp.int32),
            pltpu.VMEM((KB * EB,), jnp.float32),
            pltpu.VMEM((U_pad // 2,), jnp.int32),
            pltpu.VMEM((EB,), jnp.int32),
            pltpu.VMEM((EB,), jnp.int32),
            pltpu.VMEM((EB, DH), jnp.float32),
            pltpu.VMEM((EB, DH), jnp.float32),
            pltpu.VMEM((1, EB), jnp.int32),
            pltpu.VMEM_SHARED((B_pad, DH), jnp.float32),
            pltpu.SemaphoreType.DMA,
            pltpu.SemaphoreType.DMA,
            pltpu.SemaphoreType.DMA,
        ],
        compiler_params=pltpu.CompilerParams(needs_layout_passes=False),
    )
    planes = sc_call(nodes_p, uniq16, rows_p, cols_p, vals_p, ftab2)

    W4 = W.reshape(4, DH, EMB)
    b2 = b.reshape(1, EMB)
    RT = 512
    mm = pl.pallas_call(
        _mm_body,
        grid=(B_pad // RT,),
        in_specs=[
            pl.BlockSpec((4, RT, DH), lambda i: (0, i, 0)),
            pl.BlockSpec((4, DH, EMB), lambda i: (0, 0, 0)),
            pl.BlockSpec((1, EMB), lambda i: (0, 0)),
        ],
        out_specs=pl.BlockSpec((RT, EMB), lambda i: (i, 0)),
        out_shape=jax.ShapeDtypeStruct((B, EMB), jnp.float32),
    )
    return mm(planes, W4, b2)


# speedup vs baseline: 3.2107x; 1.0101x over previous
import functools

import jax
import jax.numpy as jnp
from jax import lax
from jax.experimental import pallas as pl
from jax.experimental.pallas import tpu as pltpu
from jax.experimental.pallas import tpu_sc as plsc

NC = 2
NS = 16
LANES = 16

EB = 128
KB = 8


def _sc_body(nb, nodes_h, uniq_h, rows_h, cols_h, vals_h, ftab_h, out_h,
             rows_st, colsA_st, colsB_st, vals_st, uniq_v,
             effbA, effbB, bufA, bufB, idxs, acc, semA, semB, semG):
    c = lax.axis_index("c")
    s = lax.axis_index("s")
    rpt = acc.shape[0] // NS
    nck = rpt // EB

    base = s * nb
    kb = rows_st.shape[0]
    nsb = nb // kb

    pltpu.sync_copy(uniq_h, uniq_v)

    def compute_eff(cols_ref, lb, dst):
        for g in range(EB // LANES):
            sl = pl.ds(g * LANES, LANES)
            cv = cols_ref[lb, sl]
            pair = plsc.load_gather(uniq_v, [lax.shift_right_logical(cv, 1)])
            sh = (cv & 1) << 4
            u = lax.shift_right_logical(pair, sh) & 0xFFFF
            dst[sl] = u * 2 + c

    zero16 = jnp.zeros((LANES,), jnp.float32)

    def zrow(i, carry):
        for j in range(8):
            bufA[i, pl.ds(j * LANES, LANES)] = zero16
        return carry

    lax.fori_loop(0, EB, zrow, 0)
    for k in range(nck):
        pltpu.sync_copy(bufA, acc.at[pl.ds(rpt * s + EB * k, EB)])

    for k in range(nck):
        pltpu.sync_copy(nodes_h.at[s, k], idxs)
        for j in range(8):
            sl = pl.ds(j * LANES, LANES)
            idxs[0, sl] = idxs[0, sl] * 2 + c
        pltpu.async_copy(ftab_h.at[idxs.at[0]], bufB, semG).wait()
        pltpu.sync_copy(bufB, out_h.at[2 + c, pl.ds(rpt * s + EB * k, EB)])

    pltpu.sync_copy(cols_h.at[pl.ds(base, kb)], colsA_st)
    compute_eff(colsA_st, 0, effbA)
    pltpu.async_copy(ftab_h.at[effbA], bufA, semA)

    plsc.subcore_barrier()

    def piter(i, carry):
        li = lax.rem(i, 4)
        t = i // 4
        p = lax.rem(t, 2)
        lb0 = 2 * li

        @pl.when((li == 0) & (i == 0))
        def _stage():
            pltpu.sync_copy(rows_h.at[pl.ds(base + t * kb, kb)], rows_st)
            pltpu.sync_copy(vals_h.at[pl.ds((base + t * kb) * EB, kb * EB)],
                            vals_st)

        @pl.when((li == 0) & (i > 0))
        def _stage_wait():
            pltpu.make_async_copy(rows_h.at[pl.ds(base + t * kb, kb)],
                                  rows_st, semG).wait()
            pltpu.make_async_copy(vals_h.at[pl.ds((base + t * kb) * EB,
                                                  kb * EB)],
                                  vals_st, semG).wait()

        @pl.when(p == 0)
        def _eb0():
            compute_eff(colsA_st, lb0 + 1, effbB)

        @pl.when(p == 1)
        def _eb1():
            compute_eff(colsB_st, lb0 + 1, effbB)

        pltpu.async_copy(ftab_h.at[effbB], bufB, semB)

        @pl.when((li == 2) & (t + 1 < nsb))
        def _pf():
            @pl.when(p == 0)
            def _pf0():
                pltpu.async_copy(cols_h.at[pl.ds(base + (t + 1) * kb, kb)],
                                 colsB_st, semG)

            @pl.when(p == 1)
            def _pf1():
                pltpu.async_copy(cols_h.at[pl.ds(base + (t + 1) * kb, kb)],
                                 colsA_st, semG)

        pltpu.make_async_copy(ftab_h.at[effbA], bufA, semA).wait()

        def egrpA(g, cc):
            wv = vals_st[pl.ds(lb0 * EB + g * LANES, LANES)]
            for u in range(LANES):
                w = wv[u]
                i2 = g * LANES + u
                for j in range(8):
                    sl = pl.ds(j * LANES, LANES)
                    bufA[i2, sl] = bufA[i2, sl] * w
            return cc

        lax.fori_loop(0, EB // LANES, egrpA, 0)
        pltpu.async_copy(bufA, acc.at[rows_st.at[lb0]], semA, add=True)
        pltpu.make_async_copy(ftab_h.at[effbA], bufA, semA).wait()

        @pl.when(i < nb // 2 - 1)
        def _next():
            @pl.when(li < 3)
            def _n_same():
                @pl.when(p == 0)
                def _ns0():
                    compute_eff(colsA_st, lb0 + 2, effbA)

                @pl.when(p == 1)
                def _ns1():
                    compute_eff(colsB_st, lb0 + 2, effbA)

            @pl.when(li == 3)
            def _n_cross():
                @pl.when(p == 0)
                def _nc0():
                    pltpu.make_async_copy(
                        cols_h.at[pl.ds(base + (t + 1) * kb, kb)],
                        colsB_st, semG).wait()
                    compute_eff(colsB_st, 0, effbA)

                @pl.when(p == 1)
                def _nc1():
                    pltpu.make_async_copy(
                        cols_h.at[pl.ds(base + (t + 1) * kb, kb)],
                        colsA_st, semG).wait()
                    compute_eff(colsA_st, 0, effbA)

            pltpu.async_copy(ftab_h.at[effbA], bufA, semA)

        pltpu.make_async_copy(ftab_h.at[effbB], bufB, semB).wait()

        def egrpB(g, cc):
            wv = vals_st[pl.ds((lb0 + 1) * EB + g * LANES, LANES)]
            for u in range(LANES):
                w = wv[u]
                i2 = g * LANES + u
                for j in range(8):
                    sl = pl.ds(j * LANES, LANES)
                    bufB[i2, sl] = bufB[i2, sl] * w
            return cc

        lax.fori_loop(0, EB // LANES, egrpB, 0)
        pltpu.async_copy(bufB, acc.at[rows_st.at[lb0 + 1]], semB, add=True)
        pltpu.make_async_copy(ftab_h.at[effbB], bufB, semB).wait()

        @pl.when((li == 3) & (t + 1 < nsb))
        def _pf_rv():
            pltpu.async_copy(rows_h.at[pl.ds(base + (t + 1) * kb, kb)],
                             rows_st, semG)
            pltpu.async_copy(vals_h.at[pl.ds((base + (t + 1) * kb) * EB,
                                             kb * EB)],
                             vals_st, semG)
        return carry

    lax.fori_loop(0, nb // 2, piter, 0)

    plsc.subcore_barrier()
    for k in range(nck):
        pltpu.sync_copy(acc.at[pl.ds(rpt * s + EB * k, EB)], bufB)
        pltpu.sync_copy(bufB, out_h.at[c, pl.ds(rpt * s + EB * k, EB)])


def _prep_body(mi_ref, mv_ref, r_ref, c_ref, v_ref):
    n = mi_ref.shape[1]
    pad = r_ref.shape[0] - n
    r_ref[pl.ds(0, n)] = mi_ref[0, :]
    c_ref[pl.ds(0, n)] = mi_ref[1, :]
    v_ref[pl.ds(0, n)] = mv_ref[...]
    r_ref[pl.ds(n, pad)] = jnp.zeros((pad,), jnp.int32)
    c_ref[pl.ds(n, pad)] = jnp.zeros((pad,), jnp.int32)
    v_ref[pl.ds(n, pad)] = jnp.zeros((pad,), jnp.float32)


def _mm_body(x_ref, w_ref, b_ref, o_ref):
    acc = jnp.dot(x_ref[0], w_ref[0], preferred_element_type=jnp.float32)
    for k in range(1, 4):
        acc += jnp.dot(x_ref[k], w_ref[k], preferred_element_type=jnp.float32)
    acc += b_ref[...]
    o_ref[...] = acc * jax.nn.sigmoid(acc)


def kernel(nodes, mask_indices, mask_values, unique_nodes_list, feature_table, W, b):
    B = nodes.shape[0]
    NNZ = mask_values.shape[0]
    NTAB, D = feature_table.shape
    EMB = W.shape[1]
    DH = D // 2

    B_pad = ((B + NS * EB - 1) // (NS * EB)) * (NS * EB)
    NNZ_pad = ((NNZ + NS * EB * 8 - 1) // (NS * EB * 8)) * (NS * EB * 8)
    nb = NNZ_pad // NS // EB
    rpt = B_pad // NS
    nck = rpt // EB

    prep = pl.pallas_call(
        _prep_body,
        out_shape=(
            jax.ShapeDtypeStruct((NNZ_pad,), jnp.int32),
            jax.ShapeDtypeStruct((NNZ_pad,), jnp.int32),
            jax.ShapeDtypeStruct((NNZ_pad,), jnp.float32),
        ),
    )
    rows_f, cols_f, vals_p = prep(mask_indices, mask_values)
    rows_p = rows_f.reshape(NNZ_pad // EB, EB)
    cols_p = cols_f.reshape(NNZ_pad // EB, EB)
    nodes_p = jnp.concatenate(
        [nodes, jnp.zeros((B_pad - B,), jnp.int32)]
    ).reshape(NS, nck, 1, EB)
    ftab2 = feature_table.reshape(NTAB * 2, DH)

    U = unique_nodes_list.shape[0]
    U_pad = ((U + 2 * LANES - 1) // (2 * LANES)) * (2 * LANES)
    up = jnp.concatenate(
        [unique_nodes_list, jnp.zeros((U_pad - U,), jnp.int32)]
    ).astype(jnp.uint16)
    uniq16 = lax.bitcast_convert_type(up.reshape(U_pad // 2, 2), jnp.int32)

    mesh = plsc.VectorSubcoreMesh(
        core_axis_name="c", subcore_axis_name="s", num_cores=NC, num_subcores=NS
    )
    sc_call = pl.kernel(
        functools.partial(_sc_body, nb),
        out_type=jax.ShapeDtypeStruct((4, B_pad, DH), jnp.float32),
        mesh=mesh,
        scratch_types=[
            pltpu.VMEM((KB, EB), jnp.int32),
            pltpu.VMEM((KB, EB), jnp.int32),
            pltpu.VMEM((KB, EB), jnp.int32),
            pltpu.VMEM((KB * EB,), jnp.float32),
            pltpu.VMEM((U_pad // 2,), jnp.int32),
            pltpu.VMEM((EB,), jnp.int32),
            pltpu.VMEM((EB,), jnp.int32),
            pltpu.VMEM((EB, DH), jnp.float32),
            pltpu.VMEM((EB, DH), jnp.float32),
            pltpu.VMEM((1, EB), jnp.int32),
            pltpu.VMEM_SHARED((B_pad, DH), jnp.float32),
            pltpu.SemaphoreType.DMA,
            pltpu.SemaphoreType.DMA,
            pltpu.SemaphoreType.DMA,
        ],
        compiler_params=pltpu.CompilerParams(needs_layout_passes=False),
    )
    planes = sc_call(nodes_p, uniq16, rows_p, cols_p, vals_p, ftab2)

    W4 = W.reshape(4, DH, EMB)
    b2 = b.reshape(1, EMB)
    RT = 512
    mm = pl.pallas_call(
        _mm_body,
        grid=(B_pad // RT,),
        in_specs=[
            pl.BlockSpec((4, RT, DH), lambda i: (0, i, 0)),
            pl.BlockSpec((4, DH, EMB), lambda i: (0, 0, 0)),
            pl.BlockSpec((1, EMB), lambda i: (0, 0)),
        ],
        out_specs=pl.BlockSpec((RT, EMB), lambda i: (i, 0)),
        out_shape=jax.ShapeDtypeStruct((B, EMB), jnp.float32),
    )
    return mm(planes, W4, b2)


# speedup vs baseline: 3.2405x; 1.0093x over previous
import functools

import jax
import jax.numpy as jnp
from jax import lax
from jax.experimental import pallas as pl
from jax.experimental.pallas import tpu as pltpu
from jax.experimental.pallas import tpu_sc as plsc

NC = 2
NS = 16
LANES = 16

EB = 128
KB = 8


def _sc_body(nb, nodes_h, uniq_h, rows_h, cols_h, vals_h, ftab_h, out_h,
             rows_st, colsA_st, colsB_st, vals_st, uniq_v,
             effbA, effbB, bufA, bufB, idxs, idxs2, acc, semA, semB, semG):
    c = lax.axis_index("c")
    s = lax.axis_index("s")
    rpt = acc.shape[0] // NS
    nck = rpt // EB

    base = s * nb
    kb = rows_st.shape[0]
    nsb = nb // kb

    pltpu.sync_copy(uniq_h, uniq_v)

    def compute_eff(cols_ref, lb, dst):
        for g in range(EB // LANES):
            sl = pl.ds(g * LANES, LANES)
            cv = cols_ref[lb, sl]
            pair = plsc.load_gather(uniq_v, [lax.shift_right_logical(cv, 1)])
            sh = (cv & 1) << 4
            u = lax.shift_right_logical(pair, sh) & 0xFFFF
            dst[sl] = u * 2 + c

    ibufs = [idxs, idxs2]
    gbufs = [bufA, bufB]

    def _stage_idx(k):
        ib = ibufs[k % 2]
        pltpu.sync_copy(nodes_h.at[s, k], ib)
        for j in range(8):
            sl = pl.ds(j * LANES, LANES)
            ib[0, sl] = ib[0, sl] * 2 + c

    _stage_idx(0)
    pltpu.async_copy(ftab_h.at[idxs.at[0]], bufA, semG)
    for k in range(nck):
        ib, gb = ibufs[k % 2], gbufs[k % 2]
        if k + 1 < nck:
            _stage_idx(k + 1)
        pltpu.make_async_copy(ftab_h.at[ib.at[0]], gb, semG).wait()
        if k + 1 < nck:
            pltpu.async_copy(ftab_h.at[ibufs[(k + 1) % 2].at[0]],
                             gbufs[(k + 1) % 2], semG)
        pltpu.sync_copy(gb, out_h.at[2 + c, pl.ds(rpt * s + EB * k, EB)])

    zero16 = jnp.zeros((LANES,), jnp.float32)

    def zrow(i, carry):
        for j in range(8):
            bufA[i, pl.ds(j * LANES, LANES)] = zero16
        return carry

    lax.fori_loop(0, EB, zrow, 0)
    for k in range(nck):
        pltpu.async_copy(bufA, acc.at[pl.ds(rpt * s + EB * k, EB)], semG)
    for k in range(nck):
        pltpu.make_async_copy(bufA, acc.at[pl.ds(rpt * s + EB * k, EB)],
                              semG).wait()

    pltpu.sync_copy(cols_h.at[pl.ds(base, kb)], colsA_st)
    compute_eff(colsA_st, 0, effbA)
    pltpu.async_copy(ftab_h.at[effbA], bufA, semA)

    plsc.subcore_barrier()

    def piter(i, carry):
        li = lax.rem(i, 4)
        t = i // 4
        p = lax.rem(t, 2)
        lb0 = 2 * li

        @pl.when((li == 0) & (i == 0))
        def _stage():
            pltpu.sync_copy(rows_h.at[pl.ds(base + t * kb, kb)], rows_st)
            pltpu.sync_copy(vals_h.at[pl.ds((base + t * kb) * EB, kb * EB)],
                            vals_st)

        @pl.when((li == 0) & (i > 0))
        def _stage_wait():
            pltpu.make_async_copy(rows_h.at[pl.ds(base + t * kb, kb)],
                                  rows_st, semG).wait()
            pltpu.make_async_copy(vals_h.at[pl.ds((base + t * kb) * EB,
                                                  kb * EB)],
                                  vals_st, semG).wait()

        @pl.when(p == 0)
        def _eb0():
            compute_eff(colsA_st, lb0 + 1, effbB)

        @pl.when(p == 1)
        def _eb1():
            compute_eff(colsB_st, lb0 + 1, effbB)

        pltpu.async_copy(ftab_h.at[effbB], bufB, semB)

        @pl.when((li == 2) & (t + 1 < nsb))
        def _pf():
            @pl.when(p == 0)
            def _pf0():
                pltpu.async_copy(cols_h.at[pl.ds(base + (t + 1) * kb, kb)],
                                 colsB_st, semG)

            @pl.when(p == 1)
            def _pf1():
                pltpu.async_copy(cols_h.at[pl.ds(base + (t + 1) * kb, kb)],
                                 colsA_st, semG)

        pltpu.make_async_copy(ftab_h.at[effbA], bufA, semA).wait()

        def egrpA(g, cc):
            wv = vals_st[pl.ds(lb0 * EB + g * LANES, LANES)]
            for u in range(LANES):
                w = wv[u]
                i2 = g * LANES + u
                for j in range(8):
                    sl = pl.ds(j * LANES, LANES)
                    bufA[i2, sl] = bufA[i2, sl] * w
            return cc

        lax.fori_loop(0, EB // LANES, egrpA, 0)
        pltpu.async_copy(bufA, acc.at[rows_st.at[lb0]], semA, add=True)
        pltpu.make_async_copy(ftab_h.at[effbA], bufA, semA).wait()

        @pl.when(i < nb // 2 - 1)
        def _next():
            @pl.when(li < 3)
            def _n_same():
                @pl.when(p == 0)
                def _ns0():
                    compute_eff(colsA_st, lb0 + 2, effbA)

                @pl.when(p == 1)
                def _ns1():
                    compute_eff(colsB_st, lb0 + 2, effbA)

            @pl.when(li == 3)
            def _n_cross():
                @pl.when(p == 0)
                def _nc0():
                    pltpu.make_async_copy(
                        cols_h.at[pl.ds(base + (t + 1) * kb, kb)],
                        colsB_st, semG).wait()
                    compute_eff(colsB_st, 0, effbA)

                @pl.when(p == 1)
                def _nc1():
                    pltpu.make_async_copy(
                        cols_h.at[pl.ds(base + (t + 1) * kb, kb)],
                        colsA_st, semG).wait()
                    compute_eff(colsA_st, 0, effbA)

            pltpu.async_copy(ftab_h.at[effbA], bufA, semA)

        pltpu.make_async_copy(ftab_h.at[effbB], bufB, semB).wait()

        def egrpB(g, cc):
            wv = vals_st[pl.ds((lb0 + 1) * EB + g * LANES, LANES)]
            for u in range(LANES):
                w = wv[u]
                i2 = g * LANES + u
                for j in range(8):
                    sl = pl.ds(j * LANES, LANES)
                    bufB[i2, sl] = bufB[i2, sl] * w
            return cc

        lax.fori_loop(0, EB // LANES, egrpB, 0)
        pltpu.async_copy(bufB, acc.at[rows_st.at[lb0 + 1]], semB, add=True)
        pltpu.make_async_copy(ftab_h.at[effbB], bufB, semB).wait()

        @pl.when((li == 3) & (t + 1 < nsb))
        def _pf_rv():
            pltpu.async_copy(rows_h.at[pl.ds(base + (t + 1) * kb, kb)],
                             rows_st, semG)
            pltpu.async_copy(vals_h.at[pl.ds((base + (t + 1) * kb) * EB,
                                             kb * EB)],
                             vals_st, semG)
        return carry

    lax.fori_loop(0, nb // 2, piter, 0)

    plsc.subcore_barrier()
    for k in range(nck):
        pltpu.sync_copy(acc.at[pl.ds(rpt * s + EB * k, EB)], bufB)
        pltpu.sync_copy(bufB, out_h.at[c, pl.ds(rpt * s + EB * k, EB)])


def _prep_body(mi_ref, mv_ref, r_ref, c_ref, v_ref):
    n = mi_ref.shape[1]
    pad = r_ref.shape[0] - n
    r_ref[pl.ds(0, n)] = mi_ref[0, :]
    c_ref[pl.ds(0, n)] = mi_ref[1, :]
    v_ref[pl.ds(0, n)] = mv_ref[...]
    r_ref[pl.ds(n, pad)] = jnp.zeros((pad,), jnp.int32)
    c_ref[pl.ds(n, pad)] = jnp.zeros((pad,), jnp.int32)
    v_ref[pl.ds(n, pad)] = jnp.zeros((pad,), jnp.float32)


def _mm_body(x_ref, w_ref, b_ref, o_ref):
    acc = jnp.dot(x_ref[0], w_ref[0], preferred_element_type=jnp.float32)
    for k in range(1, 4):
        acc += jnp.dot(x_ref[k], w_ref[k], preferred_element_type=jnp.float32)
    acc += b_ref[...]
    o_ref[...] = acc * jax.nn.sigmoid(acc)


def kernel(nodes, mask_indices, mask_values, unique_nodes_list, feature_table, W, b):
    B = nodes.shape[0]
    NNZ = mask_values.shape[0]
    NTAB, D = feature_table.shape
    EMB = W.shape[1]
    DH = D // 2

    B_pad = ((B + NS * EB - 1) // (NS * EB)) * (NS * EB)
    NNZ_pad = ((NNZ + NS * EB * 8 - 1) // (NS * EB * 8)) * (NS * EB * 8)
    nb = NNZ_pad // NS // EB
    rpt = B_pad // NS
    nck = rpt // EB

    prep = pl.pallas_call(
        _prep_body,
        out_shape=(
            jax.ShapeDtypeStruct((NNZ_pad,), jnp.int32),
            jax.ShapeDtypeStruct((NNZ_pad,), jnp.int32),
            jax.ShapeDtypeStruct((NNZ_pad,), jnp.float32),
        ),
    )
    rows_f, cols_f, vals_p = prep(mask_indices, mask_values)
    rows_p = rows_f.reshape(NNZ_pad // EB, EB)
    cols_p = cols_f.reshape(NNZ_pad // EB, EB)
    nodes_p = jnp.concatenate(
        [nodes, jnp.zeros((B_pad - B,), jnp.int32)]
    ).reshape(NS, nck, 1, EB)
    ftab2 = feature_table.reshape(NTAB * 2, DH)

    U = unique_nodes_list.shape[0]
    U_pad = ((U + 2 * LANES - 1) // (2 * LANES)) * (2 * LANES)
    up = jnp.concatenate(
        [unique_nodes_list, jnp.zeros((U_pad - U,), jnp.int32)]
    ).astype(jnp.uint16)
    uniq16 = lax.bitcast_convert_type(up.reshape(U_pad // 2, 2), jnp.int32)

    mesh = plsc.VectorSubcoreMesh(
        core_axis_name="c", subcore_axis_name="s", num_cores=NC, num_subcores=NS
    )
    sc_call = pl.kernel(
        functools.partial(_sc_body, nb),
        out_type=jax.ShapeDtypeStruct((4, B_pad, DH), jnp.float32),
        mesh=mesh,
        scratch_types=[
            pltpu.VMEM((KB, EB), jnp.int32),
            pltpu.VMEM((KB, EB), jnp.int32),
            pltpu.VMEM((KB, EB), jnp.int32),
            pltpu.VMEM((KB * EB,), jnp.float32),
            pltpu.VMEM((U_pad // 2,), jnp.int32),
            pltpu.VMEM((EB,), jnp.int32),
            pltpu.VMEM((EB,), jnp.int32),
            pltpu.VMEM((EB, DH), jnp.float32),
            pltpu.VMEM((EB, DH), jnp.float32),
            pltpu.VMEM((1, EB), jnp.int32),
            pltpu.VMEM((1, EB), jnp.int32),
            pltpu.VMEM_SHARED((B_pad, DH), jnp.float32),
            pltpu.SemaphoreType.DMA,
            pltpu.SemaphoreType.DMA,
            pltpu.SemaphoreType.DMA,
        ],
        compiler_params=pltpu.CompilerParams(needs_layout_passes=False),
    )
    planes = sc_call(nodes_p, uniq16, rows_p, cols_p, vals_p, ftab2)

    W4 = W.reshape(4, DH, EMB)
    b2 = b.reshape(1, EMB)
    RT = 512
    mm = pl.pallas_call(
        _mm_body,
        grid=(B_pad // RT,),
        in_specs=[
            pl.BlockSpec((4, RT, DH), lambda i: (0, i, 0)),
            pl.BlockSpec((4, DH, EMB), lambda i: (0, 0, 0)),
            pl.BlockSpec((1, EMB), lambda i: (0, 0)),
        ],
        out_specs=pl.BlockSpec((RT, EMB), lambda i: (i, 0)),
        out_shape=jax.ShapeDtypeStruct((B, EMB), jnp.float32),
    )
    return mm(planes, W4, b2)
